# Initial kernel scaffold; baseline (speedup 1.0000x reference)
import jax
import jax.numpy as jnp
from jax.experimental import pallas as pl


def kernel(reads_2d, info_2d, ref_seq_2d, W_r1, b_r1, W_r2, b_r2, W_i1, b_i1, W_i2, b_i2, W_conv, b_conv, W_seq, b_seq, enc_W1, enc_W2, enc_Wc, enc_W3, W_p1, b_p1, W_p2, b_p2, W_f1, b_f1, W_f2, b_f2, ref_counts, alt_counts):
    raise NotImplementedError("write your pallas kernel here")


# speedup vs baseline: 1.1257x; 1.1257x over previous
import functools

import jax
import jax.numpy as jnp
from jax import lax
from jax.experimental import pallas as pl
from jax.experimental.pallas import tpu as pltpu

V = 2048
D_MODEL = 512
D_FFN = 1024
NUM_BLOCKS = 2

BV = 256
TR = 256


def _full(shape):
    return pl.BlockSpec(shape, lambda *_: tuple(0 for _ in shape))


def _variant_body(info_ref, patches_ref, wi1, bi1, wi2, bi2, wconv, bconv,
                  wseq, bseq, iseq_out):
    info = info_ref[...]
    e = jnp.maximum(jnp.dot(info, wi1[...], preferred_element_type=jnp.float32)
                    + bi1[...], 0.0)
    e = jnp.maximum(jnp.dot(e, wi2[...], preferred_element_type=jnp.float32)
                    + bi2[...], 0.0)
    p = patches_ref[...]
    c = jnp.dot(p, wconv[...], preferred_element_type=jnp.float32) + bconv[...]
    c = jnp.maximum(c, 0.0)
    c = c.reshape(BV, 60, 64)
    feat = jnp.max(c, axis=1)
    s = jnp.maximum(jnp.dot(feat, wseq[...], preferred_element_type=jnp.float32)
                    + bseq[...], 0.0)
    iseq_out[:, :128] = e
    iseq_out[:, 128:] = s


def _variant_stage(info_2d, patches, W_i1, b_i1, W_i2, b_i2, W_conv2d, b_conv,
                   W_seq, b_seq):
    grid = (V // BV,)
    return pl.pallas_call(
        _variant_body,
        grid=grid,
        in_specs=[
            pl.BlockSpec((BV, 64), lambda i: (i, 0)),
            pl.BlockSpec((BV * 60, 24), lambda i: (i, 0)),
            _full((64, 128)), _full((1, 128)),
            _full((128, 128)), _full((1, 128)),
            _full((24, 64)), _full((1, 64)),
            _full((64, 128)), _full((1, 128)),
        ],
        out_specs=pl.BlockSpec((BV, 256), lambda i: (i, 0)),
        out_shape=jax.ShapeDtypeStruct((V, 256), jnp.float32),
    )(info_2d, patches, W_i1, b_i1, W_i2, b_i2, W_conv2d, b_conv, W_seq, b_seq)


def _read_body(reads_ref, iseq_g_ref, w1, b1, w2, b2, x_out):
    r = reads_ref[...]
    e = jnp.maximum(jnp.dot(r, w1[...], preferred_element_type=jnp.float32)
                    + b1[...], 0.0)
    e = jnp.maximum(jnp.dot(e, w2[...], preferred_element_type=jnp.float32)
                    + b2[...], 0.0)
    x_out[:, :256] = e
    x_out[:, 256:] = iseq_g_ref[...]


def _read_stage(reads_2d, iseq_g, W_r1, b_r1, W_r2, b_r2, ntiles):
    total = reads_2d.shape[0]
    return pl.pallas_call(
        _read_body,
        grid=(ntiles,),
        in_specs=[
            pl.BlockSpec((TR, 128), lambda i: (i, 0)),
            pl.BlockSpec((TR, 256), lambda i: (i, 0)),
            _full((128, 256)), _full((1, 256)),
            _full((256, 256)), _full((1, 256)),
        ],
        out_specs=pl.BlockSpec((TR, D_MODEL), lambda i: (i, 0)),
        out_shape=jax.ShapeDtypeStruct((total, D_MODEL), jnp.float32),
    )(reads_2d, iseq_g, W_r1, b_r1, W_r2, b_r2)


def _ctx_body(sr_ref, sa_ref, rc_ref, ac_ref, wc0, wc1, ctx0_out, ctx1_out):
    ref_mean = sr_ref[...] / rc_ref[...]
    alt_mean = sa_ref[...] / ac_ref[...]
    ctx0_out[...] = jnp.dot(alt_mean, wc0[...],
                            preferred_element_type=jnp.float32)
    ctx1_out[...] = jnp.dot(ref_mean, wc1[...],
                            preferred_element_type=jnp.float32)


def _ctx_stage(sum_ref, sum_alt, rc, ac, Wc0, Wc1):
    grid = (V // BV,)
    spec = pl.BlockSpec((BV, D_MODEL), lambda i: (i, 0))
    ospec = pl.BlockSpec((BV, D_FFN), lambda i: (i, 0))
    return pl.pallas_call(
        _ctx_body,
        grid=grid,
        in_specs=[spec, spec,
                  pl.BlockSpec((BV, 1), lambda i: (i, 0)),
                  pl.BlockSpec((BV, 1), lambda i: (i, 0)),
                  _full((D_MODEL, D_FFN)), _full((D_MODEL, D_FFN))],
        out_specs=[ospec, ospec],
        out_shape=[jax.ShapeDtypeStruct((V, D_FFN), jnp.float32),
                   jax.ShapeDtypeStruct((V, D_FFN), jnp.float32)],
    )(sum_ref, sum_alt, rc, ac, Wc0, Wc1)


def _enc_body(r_smem, x_ref, ctx_ref, w10, w11, w20, w21, w30, w31, x_out):
    t = pl.program_id(0)
    rows = t * TR + lax.broadcasted_iota(jnp.int32, (TR, 1), 0)
    altm = rows >= r_smem[0]
    x = x_ref[...]
    ctx = ctx_ref[...]
    u0 = jnp.dot(x, w10[...], preferred_element_type=jnp.float32)
    u1 = jnp.dot(x, w11[...], preferred_element_type=jnp.float32)
    g0 = jnp.dot(x, w20[...], preferred_element_type=jnp.float32) + ctx
    g1 = jnp.dot(x, w21[...], preferred_element_type=jnp.float32) + ctx
    u = jnp.where(altm, u1, u0)
    g = jax.nn.sigmoid(jnp.where(altm, g1, g0))
    ug = u * g
    d0 = jnp.dot(ug, w30[...], preferred_element_type=jnp.float32)
    d1 = jnp.dot(ug, w31[...], preferred_element_type=jnp.float32)
    x_out[...] = x + jnp.where(altm, d1, d0)


def _enc_stage(R, x, ctx_g, W10, W11, W20, W21, W30, W31, ntiles):
    total = x.shape[0]
    grid_spec = pltpu.PrefetchScalarGridSpec(
        num_scalar_prefetch=1,
        grid=(ntiles,),
        in_specs=[
            pl.BlockSpec((TR, D_MODEL), lambda i, s: (i, 0)),
            pl.BlockSpec((TR, D_FFN), lambda i, s: (i, 0)),
            _full((D_MODEL, D_FFN)), _full((D_MODEL, D_FFN)),
            _full((D_MODEL, D_FFN)), _full((D_MODEL, D_FFN)),
            _full((D_FFN, D_MODEL)), _full((D_FFN, D_MODEL)),
        ],
        out_specs=pl.BlockSpec((TR, D_MODEL), lambda i, s: (i, 0)),
    )
    return pl.pallas_call(
        _enc_body,
        grid_spec=grid_spec,
        out_shape=jax.ShapeDtypeStruct((total, D_MODEL), jnp.float32),
    )(jnp.array([R], jnp.int32) if not hasattr(R, "dtype") else R.reshape(1).astype(jnp.int32),
      x, ctx_g, W10, W11, W20, W21, W30, W31)


def _phi_body(x_ref, wp1, bp1, wp2, bp2, phi_out):
    x = x_ref[...]
    p = jnp.maximum(jnp.dot(x, wp1[...], preferred_element_type=jnp.float32)
                    + bp1[...], 0.0)
    p = jnp.maximum(jnp.dot(p, wp2[...], preferred_element_type=jnp.float32)
                    + bp2[...], 0.0)
    phi_out[...] = p


def _phi_stage(x, W_p1, b_p1, W_p2, b_p2, ntiles):
    total = x.shape[0]
    return pl.pallas_call(
        _phi_body,
        grid=(ntiles,),
        in_specs=[
            pl.BlockSpec((TR, D_MODEL), lambda i: (i, 0)),
            _full((D_MODEL, 1024)), _full((1, 1024)),
            _full((1024, 1024)), _full((1, 1024)),
        ],
        out_specs=pl.BlockSpec((TR, 1024), lambda i: (i, 0)),
        out_shape=jax.ShapeDtypeStruct((total, 1024), jnp.float32),
    )(x, W_p1, b_p1, W_p2, b_p2)


def _final_body(pooled_ref, ac_ref, wf1, bf1, wf2, bf2, out_ref):
    pooled = pooled_ref[...] / ac_ref[...]
    h = jnp.maximum(jnp.dot(pooled, wf1[...], preferred_element_type=jnp.float32)
                    + bf1[...], 0.0)
    out_ref[...] = jnp.dot(h, wf2[...], preferred_element_type=jnp.float32) \
        + bf2[...]


def _final_stage(pool_sum, ac, W_f1, b_f1, W_f2, b_f2):
    grid = (V // BV,)
    return pl.pallas_call(
        _final_body,
        grid=grid,
        in_specs=[
            pl.BlockSpec((BV, 1024), lambda i: (i, 0)),
            pl.BlockSpec((BV, 1), lambda i: (i, 0)),
            _full((1024, 512)), _full((1, 512)),
            _full((512, 256)), _full((1, 256)),
        ],
        out_specs=pl.BlockSpec((BV, 256), lambda i: (i, 0)),
        out_shape=jax.ShapeDtypeStruct((V, 256), jnp.float32),
    )(pool_sum, ac, W_f1, b_f1, W_f2, b_f2)


def kernel(reads_2d, info_2d, ref_seq_2d, W_r1, b_r1, W_r2, b_r2, W_i1, b_i1,
           W_i2, b_i2, W_conv, b_conv, W_seq, b_seq, enc_W1, enc_W2, enc_Wc,
           enc_W3, W_p1, b_p1, W_p2, b_p2, W_f1, b_f1, W_f2, b_f2,
           ref_counts, alt_counts):
    total = reads_2d.shape[0]
    ntiles = (total + TR - 1) // TR

    counts2 = jnp.concatenate((ref_counts, alt_counts)).astype(jnp.int32)
    seg2 = jnp.repeat(jnp.arange(2 * V, dtype=jnp.int32), counts2,
                      total_repeat_length=total)
    is_alt = seg2 >= V
    var_all = jnp.where(is_alt, seg2 - V, seg2)
    R = jnp.sum(ref_counts).astype(jnp.int32)

    rc = ref_counts.astype(jnp.float32).reshape(V, 1)
    ac = alt_counts.astype(jnp.float32).reshape(V, 1)

    x3 = ref_seq_2d.reshape(V, 4, 64)
    cols = [x3[:, i, k:k + 60] for i in range(4) for k in range(5)]
    patches = jnp.stack(cols + [jnp.zeros((V, 60), jnp.float32)] * 4, axis=-1)
    patches = patches.reshape(V * 60, 24)
    W_conv2d = jnp.concatenate(
        [W_conv.reshape(64, 20).T, jnp.zeros((4, 64), jnp.float32)], axis=0)

    iseq = _variant_stage(info_2d, patches, W_i1, b_i1.reshape(1, -1),
                          W_i2, b_i2.reshape(1, -1), W_conv2d,
                          b_conv.reshape(1, -1), W_seq, b_seq.reshape(1, -1))
    ref_seq_embeddings_ve = iseq[:, 128:]

    iseq_g = jnp.take(iseq, var_all, axis=0)
    x = _read_stage(reads_2d, iseq_g, W_r1, b_r1.reshape(1, -1),
                    W_r2, b_r2.reshape(1, -1), ntiles)

    zero = jnp.zeros((), jnp.float32)
    alt_col = is_alt[:, None]
    for b in range(NUM_BLOCKS):
        sums = jax.ops.segment_sum(x, seg2, num_segments=2 * V)
        ctx0, ctx1 = _ctx_stage(sums[:V], sums[V:], rc, ac,
                                enc_Wc[b, 0], enc_Wc[b, 1])
        ctx_g = jnp.where(alt_col, jnp.take(ctx1, var_all, axis=0),
                          jnp.take(ctx0, var_all, axis=0))
        x = _enc_stage(R, x, ctx_g, enc_W1[b, 0], enc_W1[b, 1],
                       enc_W2[b, 0], enc_W2[b, 1],
                       enc_W3[b, 0], enc_W3[b, 1], ntiles)

    phi = _phi_stage(x, W_p1, b_p1.reshape(1, -1), W_p2, b_p2.reshape(1, -1),
                     ntiles)
    phi_alt = jnp.where(alt_col, phi, zero)
    pool_sum = jax.ops.segment_sum(phi_alt, var_all, num_segments=V)
    result_be = _final_stage(pool_sum, ac, W_f1, b_f1.reshape(1, -1),
                             W_f2, b_f2.reshape(1, -1))
    return result_be, ref_seq_embeddings_ve


# speedup vs baseline: 1.2187x; 1.0826x over previous
import functools

import jax
import jax.numpy as jnp
from jax import lax
from jax.experimental import pallas as pl
from jax.experimental.pallas import tpu as pltpu

V = 2048
D_MODEL = 512
D_FFN = 1024
NUM_BLOCKS = 2

BV = 256
TR = 256


def _full(shape):
    return pl.BlockSpec(shape, lambda *_: tuple(0 for _ in shape))


def _variant_body(info_ref, patches_ref, wi1, bi1, wi2, bi2, wconv, bconv,
                  wseq, bseq, iseq_out):
    info = info_ref[...]
    e = jnp.maximum(jnp.dot(info, wi1[...], preferred_element_type=jnp.float32)
                    + bi1[...], 0.0)
    e = jnp.maximum(jnp.dot(e, wi2[...], preferred_element_type=jnp.float32)
                    + bi2[...], 0.0)
    p = patches_ref[...]
    c = jnp.dot(p, wconv[...], preferred_element_type=jnp.float32) + bconv[...]
    c = jnp.maximum(c, 0.0)
    c = c.reshape(BV, 60, 64)
    feat = jnp.max(c, axis=1)
    s = jnp.maximum(jnp.dot(feat, wseq[...], preferred_element_type=jnp.float32)
                    + bseq[...], 0.0)
    iseq_out[:, :128] = e
    iseq_out[:, 128:] = s


def _variant_stage(info_2d, patches, W_i1, b_i1, W_i2, b_i2, W_conv2d, b_conv,
                   W_seq, b_seq):
    grid = (V // BV,)
    return pl.pallas_call(
        _variant_body,
        grid=grid,
        in_specs=[
            pl.BlockSpec((BV, 64), lambda i: (i, 0)),
            pl.BlockSpec((BV * 60, 24), lambda i: (i, 0)),
            _full((64, 128)), _full((1, 128)),
            _full((128, 128)), _full((1, 128)),
            _full((24, 64)), _full((1, 64)),
            _full((64, 128)), _full((1, 128)),
        ],
        out_specs=pl.BlockSpec((BV, 256), lambda i: (i, 0)),
        out_shape=jax.ShapeDtypeStruct((V, 256), jnp.float32),
    )(info_2d, patches, W_i1, b_i1, W_i2, b_i2, W_conv2d, b_conv, W_seq, b_seq)


def _read_body(reads_ref, iseq_g_ref, w1, b1, w2, b2, x_out):
    r = reads_ref[...]
    e = jnp.maximum(jnp.dot(r, w1[...], preferred_element_type=jnp.float32)
                    + b1[...], 0.0)
    e = jnp.maximum(jnp.dot(e, w2[...], preferred_element_type=jnp.float32)
                    + b2[...], 0.0)
    x_out[:, :256] = e
    x_out[:, 256:] = iseq_g_ref[...]


def _read_stage(reads_2d, iseq_g, W_r1, b_r1, W_r2, b_r2, ntiles):
    total = reads_2d.shape[0]
    return pl.pallas_call(
        _read_body,
        grid=(ntiles,),
        in_specs=[
            pl.BlockSpec((TR, 128), lambda i: (i, 0)),
            pl.BlockSpec((TR, 256), lambda i: (i, 0)),
            _full((128, 256)), _full((1, 256)),
            _full((256, 256)), _full((1, 256)),
        ],
        out_specs=pl.BlockSpec((TR, D_MODEL), lambda i: (i, 0)),
        out_shape=jax.ShapeDtypeStruct((total, D_MODEL), jnp.float32),
    )(reads_2d, iseq_g, W_r1, b_r1, W_r2, b_r2)


def _ctx_body(sr_ref, sa_ref, rc_ref, ac_ref, wc0, wc1, ctx0_out, ctx1_out):
    ref_mean = sr_ref[...] / rc_ref[...]
    alt_mean = sa_ref[...] / ac_ref[...]
    ctx0_out[...] = jnp.dot(alt_mean, wc0[...],
                            preferred_element_type=jnp.float32)
    ctx1_out[...] = jnp.dot(ref_mean, wc1[...],
                            preferred_element_type=jnp.float32)


def _ctx_stage(sum_ref, sum_alt, rc, ac, Wc0, Wc1):
    grid = (V // BV,)
    spec = pl.BlockSpec((BV, D_MODEL), lambda i: (i, 0))
    ospec = pl.BlockSpec((BV, D_FFN), lambda i: (i, 0))
    return pl.pallas_call(
        _ctx_body,
        grid=grid,
        in_specs=[spec, spec,
                  pl.BlockSpec((BV, 1), lambda i: (i, 0)),
                  pl.BlockSpec((BV, 1), lambda i: (i, 0)),
                  _full((D_MODEL, D_FFN)), _full((D_MODEL, D_FFN))],
        out_specs=[ospec, ospec],
        out_shape=[jax.ShapeDtypeStruct((V, D_FFN), jnp.float32),
                   jax.ShapeDtypeStruct((V, D_FFN), jnp.float32)],
    )(sum_ref, sum_alt, rc, ac, Wc0, Wc1)


def _enc_body(tix_s, wsel_s, init_s, r_s, x_ref, ctx_ref, w1, w2, w3, x_out):
    g = pl.program_id(0)
    rows = tix_s[g] * TR + lax.broadcasted_iota(jnp.int32, (TR, 1), 0)
    m = jnp.logical_xor(rows < r_s[0], wsel_s[g] == 1)
    x = x_ref[...]
    u = jnp.dot(x, w1[0], preferred_element_type=jnp.float32)
    gg = jax.nn.sigmoid(jnp.dot(x, w2[0], preferred_element_type=jnp.float32)
                        + ctx_ref[...])
    d = jnp.dot(u * gg, w3[0], preferred_element_type=jnp.float32)
    d = jnp.where(m, d, 0.0)

    @pl.when(init_s[g] == 1)
    def _():
        x_out[...] = x + d

    @pl.when(init_s[g] == 0)
    def _():
        x_out[...] = x_out[...] + d


def _enc_stage(tix, wsel, init, Rarr, x, ctx_g, W1b, W2b, W3b, nsteps):
    total = x.shape[0]
    rd = lambda g, tix, wsel, init, r: (tix[g], 0)
    wt = lambda g, tix, wsel, init, r: (wsel[g], 0, 0)
    grid_spec = pltpu.PrefetchScalarGridSpec(
        num_scalar_prefetch=4,
        grid=(nsteps,),
        in_specs=[
            pl.BlockSpec((TR, D_MODEL), rd),
            pl.BlockSpec((TR, D_FFN), rd),
            pl.BlockSpec((1, D_MODEL, D_FFN), wt),
            pl.BlockSpec((1, D_MODEL, D_FFN), wt),
            pl.BlockSpec((1, D_FFN, D_MODEL), wt),
        ],
        out_specs=pl.BlockSpec((TR, D_MODEL), rd),
    )
    return pl.pallas_call(
        _enc_body,
        grid_spec=grid_spec,
        out_shape=jax.ShapeDtypeStruct((total, D_MODEL), jnp.float32),
    )(tix, wsel, init, Rarr, x, ctx_g, W1b, W2b, W3b)


def _phi_body(skip_s, x_ref, wp1, bp1, wp2, bp2, phi_out):
    @pl.when(skip_s[pl.program_id(0)] == 0)
    def _():
        x = x_ref[...]
        p = jnp.maximum(jnp.dot(x, wp1[...],
                                preferred_element_type=jnp.float32)
                        + bp1[...], 0.0)
        p = jnp.maximum(jnp.dot(p, wp2[...],
                                preferred_element_type=jnp.float32)
                        + bp2[...], 0.0)
        phi_out[...] = p


def _phi_stage(skip, x, W_p1, b_p1, W_p2, b_p2, ntiles):
    total = x.shape[0]
    rd = lambda i, s: (i, 0)
    grid_spec = pltpu.PrefetchScalarGridSpec(
        num_scalar_prefetch=1,
        grid=(ntiles,),
        in_specs=[
            pl.BlockSpec((TR, D_MODEL), rd),
            pl.BlockSpec((D_MODEL, 1024), lambda i, s: (0, 0)),
            pl.BlockSpec((1, 1024), lambda i, s: (0, 0)),
            pl.BlockSpec((1024, 1024), lambda i, s: (0, 0)),
            pl.BlockSpec((1, 1024), lambda i, s: (0, 0)),
        ],
        out_specs=pl.BlockSpec((TR, 1024), rd),
    )
    return pl.pallas_call(
        _phi_body,
        grid_spec=grid_spec,
        out_shape=jax.ShapeDtypeStruct((total, 1024), jnp.float32),
    )(skip, x, W_p1, b_p1, W_p2, b_p2)


def _final_body(pooled_ref, ac_ref, wf1, bf1, wf2, bf2, out_ref):
    pooled = pooled_ref[...] / ac_ref[...]
    h = jnp.maximum(jnp.dot(pooled, wf1[...], preferred_element_type=jnp.float32)
                    + bf1[...], 0.0)
    out_ref[...] = jnp.dot(h, wf2[...], preferred_element_type=jnp.float32) \
        + bf2[...]


def _final_stage(pool_sum, ac, W_f1, b_f1, W_f2, b_f2):
    grid = (V // BV,)
    return pl.pallas_call(
        _final_body,
        grid=grid,
        in_specs=[
            pl.BlockSpec((BV, 1024), lambda i: (i, 0)),
            pl.BlockSpec((BV, 1), lambda i: (i, 0)),
            _full((1024, 512)), _full((1, 512)),
            _full((512, 256)), _full((1, 256)),
        ],
        out_specs=pl.BlockSpec((BV, 256), lambda i: (i, 0)),
        out_shape=jax.ShapeDtypeStruct((V, 256), jnp.float32),
    )(pool_sum, ac, W_f1, b_f1, W_f2, b_f2)


def kernel(reads_2d, info_2d, ref_seq_2d, W_r1, b_r1, W_r2, b_r2, W_i1, b_i1,
           W_i2, b_i2, W_conv, b_conv, W_seq, b_seq, enc_W1, enc_W2, enc_Wc,
           enc_W3, W_p1, b_p1, W_p2, b_p2, W_f1, b_f1, W_f2, b_f2,
           ref_counts, alt_counts):
    total = reads_2d.shape[0]
    ntiles = (total + TR - 1) // TR

    counts2 = jnp.concatenate((ref_counts, alt_counts)).astype(jnp.int32)
    seg2 = jnp.repeat(jnp.arange(2 * V, dtype=jnp.int32), counts2,
                      total_repeat_length=total)
    is_alt = seg2 >= V
    var_all = jnp.where(is_alt, seg2 - V, seg2)
    R = jnp.sum(ref_counts).astype(jnp.int32)

    rc = ref_counts.astype(jnp.float32).reshape(V, 1)
    ac = alt_counts.astype(jnp.float32).reshape(V, 1)

    x3 = ref_seq_2d.reshape(V, 4, 64)
    cols = [x3[:, i, k:k + 60] for i in range(4) for k in range(5)]
    patches = jnp.stack(cols + [jnp.zeros((V, 60), jnp.float32)] * 4, axis=-1)
    patches = patches.reshape(V * 60, 24)
    W_conv2d = jnp.concatenate(
        [W_conv.reshape(64, 20).T, jnp.zeros((4, 64), jnp.float32)], axis=0)

    iseq = _variant_stage(info_2d, patches, W_i1, b_i1.reshape(1, -1),
                          W_i2, b_i2.reshape(1, -1), W_conv2d,
                          b_conv.reshape(1, -1), W_seq, b_seq.reshape(1, -1))
    ref_seq_embeddings_ve = iseq[:, 128:]

    iseq_g = jnp.take(iseq, var_all, axis=0)
    x = _read_stage(reads_2d, iseq_g, W_r1, b_r1.reshape(1, -1),
                    W_r2, b_r2.reshape(1, -1), ntiles)

    nsteps = ntiles + 1
    s_t = R // TR
    gidx = jnp.arange(nsteps, dtype=jnp.int32)
    tix = jnp.where(gidx <= s_t, gidx, gidx - 1).astype(jnp.int32)
    wsel = (gidx > s_t).astype(jnp.int32)
    init = jnp.where(gidx == s_t + 1, 0, 1).astype(jnp.int32)
    Rarr = R.reshape(1)
    skip = ((jnp.arange(ntiles, dtype=jnp.int32) + 1) * TR <= R).astype(jnp.int32)

    zero = jnp.zeros((), jnp.float32)
    alt_col = is_alt[:, None]
    for b in range(NUM_BLOCKS):
        sums = jax.ops.segment_sum(x, seg2, num_segments=2 * V)
        ctx0, ctx1 = _ctx_stage(sums[:V], sums[V:], rc, ac,
                                enc_Wc[b, 0], enc_Wc[b, 1])
        ctx_g = jnp.where(alt_col, jnp.take(ctx1, var_all, axis=0),
                          jnp.take(ctx0, var_all, axis=0))
        x = _enc_stage(tix, wsel, init, Rarr, x, ctx_g,
                       enc_W1[b], enc_W2[b], enc_W3[b], nsteps)

    phi = _phi_stage(skip, x, W_p1, b_p1.reshape(1, -1), W_p2,
                     b_p2.reshape(1, -1), ntiles)
    phi_alt = jnp.where(alt_col, phi, zero)
    pool_sum = jax.ops.segment_sum(phi_alt, var_all, num_segments=V)
    result_be = _final_stage(pool_sum, ac, W_f1, b_f1.reshape(1, -1),
                             W_f2, b_f2.reshape(1, -1))
    return result_be, ref_seq_embeddings_ve
